# baseline (speedup 1.0000x reference)
import functools

import jax
import jax.numpy as jnp
from jax import lax
from jax.experimental import pallas as pl
from jax.experimental.pallas import tpu as pltpu
from jax.experimental.pallas import tpu_sc as plsc

B = 16384
D = 64
H = 128
NC, NS = 2, 16
NW = NC * NS
BPW = B // NW
G = 128
PASS = 256


@functools.cache
def _build_sc_gather():
    mesh = plsc.VectorSubcoreMesh(
        core_axis_name="c", subcore_axis_name="s", num_cores=NC, num_subcores=NS
    )

    @functools.partial(
        pl.kernel,
        out_type=(
            jax.ShapeDtypeStruct((B, H), jnp.float32),
            jax.ShapeDtypeStruct((B, H), jnp.float32),
        ),
        mesh=mesh,
        compiler_params=pltpu.CompilerParams(needs_layout_passes=False),
        scratch_types=[
            pltpu.VMEM((BPW,), jnp.int32),
            pltpu.VMEM((BPW,), jnp.int32),
            pltpu.VMEM((PASS, H), jnp.float32),
            pltpu.VMEM((PASS, H), jnp.float32),
            pltpu.SemaphoreType.DMA,
            pltpu.SemaphoreType.DMA,
        ],
    )
    def sc_gather(uidx_hbm, bidx_hbm, utab_hbm, btab_hbm,
                  uout_hbm, bout_hbm,
                  uidx_v, bidx_v, urows, brows, semu, semb):
        wid = lax.axis_index("s") * NC + lax.axis_index("c")
        base = wid * BPW
        pltpu.sync_copy(uidx_hbm.at[pl.ds(base, BPW)], uidx_v)
        pltpu.sync_copy(bidx_hbm.at[pl.ds(base, BPW)], bidx_v)

        for p in range(BPW // PASS):
            copies = []
            for j in range(PASS // G):
                off = p * PASS + j * G
                copies.append(pltpu.async_copy(
                    utab_hbm.at[uidx_v.at[pl.ds(off, G)]],
                    urows.at[pl.ds(j * G, G)], semu))
                copies.append(pltpu.async_copy(
                    btab_hbm.at[bidx_v.at[pl.ds(off, G)]],
                    brows.at[pl.ds(j * G, G)], semb))
            for c in copies:
                c.wait()
            pltpu.sync_copy(urows, uout_hbm.at[pl.ds(base + p * PASS, PASS)])
            pltpu.sync_copy(brows, bout_hbm.at[pl.ds(base + p * PASS, PASS)])

    return sc_gather


BLK = 2048


def _tc_body(tag_ref, uf_ref, bf_ref, up_ref, bp_ref, w_ref, b_ref, out_ref):
    uf = uf_ref[...]
    bf = bf_ref[...]
    u = jnp.where(up_ref[...] == 1, uf[:, D:], uf[:, :D])
    bk = jnp.where(bp_ref[...] == 1, bf[:, D:], bf[:, :D])
    proj = jnp.dot(tag_ref[...], w_ref[...],
                   preferred_element_type=jnp.float32) + b_ref[...]
    out_ref[...] = jnp.sum(u * (bk + proj), axis=1)


def _tc_combine(tag, u_sup, b_sup, u_par, b_par, w_lin, b2d):
    return pl.pallas_call(
        _tc_body,
        grid=(B // BLK,),
        in_specs=[
            pl.BlockSpec((BLK, H), lambda i: (i, 0)),
            pl.BlockSpec((BLK, H), lambda i: (i, 0)),
            pl.BlockSpec((BLK, H), lambda i: (i, 0)),
            pl.BlockSpec((BLK, 1), lambda i: (i, 0)),
            pl.BlockSpec((BLK, 1), lambda i: (i, 0)),
            pl.BlockSpec((H, D), lambda i: (0, 0)),
            pl.BlockSpec((1, D), lambda i: (0, 0)),
        ],
        out_specs=pl.BlockSpec((BLK,), lambda i: (i,)),
        out_shape=jax.ShapeDtypeStruct((B,), jnp.float32),
    )(tag, u_sup, b_sup, u_par, b_par, w_lin, b2d)


def kernel(user, book, tag_embedding, user_table, book_table, W_lin, b_lin):
    u_super = lax.shift_right_logical(user, 1)
    b_super = lax.shift_right_logical(book, 1)
    u_par = lax.bitwise_and(user, 1).reshape(B, 1)
    b_par = lax.bitwise_and(book, 1).reshape(B, 1)
    utab2 = user_table.reshape(-1, H)
    btab2 = book_table.reshape(-1, H)
    u_sup, b_sup = _build_sc_gather()(u_super, b_super, utab2, btab2)
    return _tc_combine(tag_embedding, u_sup, b_sup, u_par, b_par,
                       W_lin, b_lin.reshape(1, D))
